# Initial kernel scaffold; baseline (speedup 1.0000x reference)
import jax
import jax.numpy as jnp
from jax.experimental import pallas as pl


def kernel(node_feat, edge_feat, edge_index, c1_We, c1_be, c1_Wr1, c1_br1, c1_Wr2, c1_br2, c1_Wl, c1_bl, c2_We, c2_be, c2_Wr1, c2_br1, c2_Wr2, c2_br2, c2_Wl, c2_bl):
    raise NotImplementedError("write your pallas kernel here")


# speedup vs baseline: 1.8619x; 1.8619x over previous
import functools

import jax
import jax.numpy as jnp
from jax import lax
from jax.experimental import pallas as pl
from jax.experimental.pallas import tpu as pltpu
from jax.experimental.pallas import tpu_sc as plsc

N = 10000
NP = 10240
E = 320000
NC = 2
NS = 16
NW = NC * NS
GB = 80
RT = NP // NS
EW = E // NW
IT = EW // GB

F32 = jnp.float32


def _zero_fill(buf, width):
    def zrow(i, carry):
        def zcol(j, carry2):
            buf[i, pl.ds(j * 16, 16)] = jnp.zeros((16,), F32)
            return carry2
        return lax.fori_loop(0, width // 16, zcol, carry)
    lax.fori_loop(0, buf.shape[0], zrow, 0)


def _ones_fill(buf, width):
    def orow(i, carry):
        def ocol(j, carry2):
            buf[i, pl.ds(j * 16, 16)] = jnp.ones((16,), F32)
            return carry2
        return lax.fori_loop(0, width // 16, ocol, carry)
    lax.fori_loop(0, buf.shape[0], orow, 0)


def _make_gather(Dv):
    mesh = plsc.VectorSubcoreMesh(
        core_axis_name="c", subcore_axis_name="s",
        num_cores=NC, num_subcores=NS)

    @functools.partial(
        pl.kernel, mesh=mesh,
        out_type=[jax.ShapeDtypeStruct((E, Dv), F32),
                  jax.ShapeDtypeStruct((E, Dv), F32)],
        scratch_types=[pltpu.VMEM((GB,), jnp.int32),
                       pltpu.VMEM((GB,), jnp.int32),
                       pltpu.VMEM((GB, Dv), F32),
                       pltpu.VMEM((GB, Dv), F32),
                       pltpu.SemaphoreType.DMA,
                       pltpu.SemaphoreType.DMA],
    )
    def k(table, src, dst, hs_out, hd_out, si, di, rs, rd, sem_s, sem_d):
        wid = lax.axis_index("s") * NC + lax.axis_index("c")
        base = wid * EW

        def body(i, carry):
            e0 = base + i * GB
            pltpu.sync_copy(src.at[pl.ds(e0, GB)], si)
            pltpu.sync_copy(dst.at[pl.ds(e0, GB)], di)
            cs = pltpu.async_copy(table.at[si], rs, sem_s)
            cd = pltpu.async_copy(table.at[di], rd, sem_d)
            cs.wait()
            cd.wait()
            pltpu.sync_copy(rs, hs_out.at[pl.ds(e0, GB)])
            pltpu.sync_copy(rd, hd_out.at[pl.ds(e0, GB)])
            return carry

        lax.fori_loop(0, IT, body, 0)

    return k


HALF = NP // 2
APAD = HALF + 128
ZR = APAD // NS
DR = HALF // NS


def _make_scatter1():
    mesh = plsc.VectorSubcoreMesh(
        core_axis_name="c", subcore_axis_name="s",
        num_cores=NC, num_subcores=NS)
    EW1 = E // NS
    IT1 = EW1 // GB

    @functools.partial(
        pl.kernel, mesh=mesh,
        out_type=[jax.ShapeDtypeStruct((NP, 128), F32),
                  jax.ShapeDtypeStruct((NP, 128), F32)],
        scratch_types=[pltpu.VMEM((GB,), jnp.int32),
                       pltpu.VMEM((GB, 128), F32),
                       pltpu.VMEM((ZR, 128), F32),
                       pltpu.VMEM_SHARED((APAD, 128), F32)],
    )
    def k(r, dst, s_out, deg_out, idxv, rbuf, zbuf, acc):
        c = lax.axis_index("c")
        s = lax.axis_index("s")
        base = s * EW1
        lo = c * HALF

        _zero_fill(zbuf, 128)
        pltpu.sync_copy(zbuf, acc.at[pl.ds(s * ZR, ZR)])
        plsc.subcore_barrier()

        def body(i, carry):
            e0 = base + i * GB
            pltpu.sync_copy(dst.at[pl.ds(e0, GB)], idxv)
            pltpu.sync_copy(r.at[pl.ds(e0, GB)], rbuf)
            for q in range(GB // 16):
                v = idxv[pl.ds(q * 16, 16)] - lo
                oob = (v < 0) | (v >= HALF)
                idxv[pl.ds(q * 16, 16)] = jnp.where(oob, HALF, v)
            pltpu.sync_copy(rbuf, acc.at[idxv], add=True)
            return carry

        lax.fori_loop(0, IT1, body, 0)
        plsc.subcore_barrier()
        pltpu.sync_copy(acc.at[pl.ds(s * DR, DR)],
                        s_out.at[pl.ds(lo + s * DR, DR)])
        plsc.subcore_barrier()

        pltpu.sync_copy(zbuf, acc.at[pl.ds(s * ZR, ZR)])
        _ones_fill(rbuf, 128)
        plsc.subcore_barrier()

        def dbody(i, carry):
            e0 = base + i * GB
            pltpu.sync_copy(dst.at[pl.ds(e0, GB)], idxv)
            for q in range(GB // 16):
                v = idxv[pl.ds(q * 16, 16)] - lo
                oob = (v < 0) | (v >= HALF)
                idxv[pl.ds(q * 16, 16)] = jnp.where(oob, HALF, v)
            pltpu.sync_copy(rbuf, acc.at[idxv], add=True)
            return carry

        lax.fori_loop(0, IT1, dbody, 0)
        plsc.subcore_barrier()
        pltpu.sync_copy(acc.at[pl.ds(s * DR, DR)],
                        deg_out.at[pl.ds(lo + s * DR, DR)])

    return k


QTR = NP // 4
QPAD = QTR + 128
QZR = QPAD // NS
QDR = QTR // NS


def _make_scatter_p2():
    mesh = plsc.VectorSubcoreMesh(
        core_axis_name="c", subcore_axis_name="s",
        num_cores=NC, num_subcores=NS)
    EW1 = E // NS
    IT1 = EW1 // GB

    @functools.partial(
        pl.kernel, mesh=mesh,
        out_type=[jax.ShapeDtypeStruct((NP, 128), F32)],
        scratch_types=[pltpu.VMEM((GB,), jnp.int32),
                       pltpu.VMEM((GB, 128), F32),
                       pltpu.VMEM((QZR, 128), F32),
                       pltpu.VMEM_SHARED((QPAD, 128), F32)],
    )
    def k(p, dst, s_out, idxv, rbuf, zbuf, acc):
        c = lax.axis_index("c")
        s = lax.axis_index("s")
        base = s * EW1

        _zero_fill(zbuf, 128)
        for j in range(2):
            lo = (2 * c + j) * QTR
            pltpu.sync_copy(zbuf, acc.at[pl.ds(s * QZR, QZR)])
            plsc.subcore_barrier()

            def body(i, carry):
                e0 = base + i * GB
                pltpu.sync_copy(dst.at[pl.ds(e0, GB)], idxv)
                pltpu.sync_copy(p.at[pl.ds(e0, GB)], rbuf)
                for q in range(GB // 16):
                    v = idxv[pl.ds(q * 16, 16)] - lo
                    oob = (v < 0) | (v >= QTR)
                    idxv[pl.ds(q * 16, 16)] = jnp.where(oob, QTR, v)
                pltpu.sync_copy(rbuf, acc.at[idxv], add=True)
                return carry

            lax.fori_loop(0, IT1, body, 0)
            plsc.subcore_barrier()
            pltpu.sync_copy(acc.at[pl.ds(s * QDR, QDR)],
                            s_out.at[pl.ds(lo + s * QDR, QDR)])
            plsc.subcore_barrier()

    return k


BE = 2000


def _cos_mlp(hs, hd, w, W1T, b1, W2T, b2):
    ns = jnp.maximum(jnp.sqrt(jnp.sum(hs * hs, axis=1, keepdims=True)), 1e-12)
    nd = jnp.maximum(jnp.sqrt(jnp.sum(hd * hd, axis=1, keepdims=True)), 1e-12)
    cos = jnp.sum(hs * hd, axis=1, keepdims=True) / (ns * nd)
    m = cos * w * hs
    rr = jnp.maximum(jnp.dot(m, W1T, preferred_element_type=F32) + b1, 0.0)
    rr = jnp.maximum(jnp.dot(rr, W2T, preferred_element_type=F32) + b2, 0.0)
    return rr


def _edge1_body(ef_ref, hs_ref, hd_ref, WeT_ref, be_ref, W1T_ref, b1_ref,
                W2T_ref, b2_ref, r_ref, w_ref):
    ef = ef_ref[...]
    WeT = WeT_ref[...]
    w = (ef[:, 0:1] * WeT[0:1, :] + ef[:, 1:2] * WeT[1:2, :]) + be_ref[...]
    r_ref[...] = _cos_mlp(hs_ref[...], hd_ref[...], w,
                          W1T_ref[...], b1_ref[...], W2T_ref[...], b2_ref[...])
    w_ref[...] = w


def _edge2_body(ef_ref, hs_ref, hd_ref, WeT_ref, be_ref, W1T_ref, b1_ref,
                W2T_ref, b2_ref, WpT_ref, p_ref):
    ef = jnp.maximum(ef_ref[...], 0.0)
    w = jnp.dot(ef, WeT_ref[...], preferred_element_type=F32) + be_ref[...]
    rr = _cos_mlp(hs_ref[...], hd_ref[...], w,
                  W1T_ref[...], b1_ref[...], W2T_ref[...], b2_ref[...])
    p_ref[...] = jnp.dot(rr, WpT_ref[...], preferred_element_type=F32)


def _full(shape):
    return pl.BlockSpec(shape, lambda i: (0,) * len(shape))


def _edge1_call(ef, hs, hd, WeT, be, W1T, b1, W2T, b2):
    Dv = hs.shape[1]
    eb = lambda last: pl.BlockSpec((BE, last), lambda i: (i, 0))
    return pl.pallas_call(
        _edge1_body,
        grid=(E // BE,),
        in_specs=[eb(2), eb(Dv), eb(Dv), _full(WeT.shape), _full(be.shape),
                  _full(W1T.shape), _full(b1.shape), _full(W2T.shape),
                  _full(b2.shape)],
        out_specs=[eb(Dv), eb(Dv)],
        out_shape=[jax.ShapeDtypeStruct((E, Dv), F32),
                   jax.ShapeDtypeStruct((E, Dv), F32)],
    )(ef, hs, hd, WeT, be, W1T, b1, W2T, b2)


def _edge2_call(ef, hs, hd, WeT, be, W1T, b1, W2T, b2, WpT):
    Dv = hs.shape[1]
    Din = ef.shape[1]
    Po = WpT.shape[1]
    eb = lambda last: pl.BlockSpec((BE, last), lambda i: (i, 0))
    return pl.pallas_call(
        _edge2_body,
        grid=(E // BE,),
        in_specs=[eb(Din), eb(Dv), eb(Dv), _full(WeT.shape), _full(be.shape),
                  _full(W1T.shape), _full(b1.shape), _full(W2T.shape),
                  _full(b2.shape), _full(WpT.shape)],
        out_specs=eb(Po),
        out_shape=jax.ShapeDtypeStruct((E, Po), F32),
    )(ef, hs, hd, WeT, be, W1T, b1, W2T, b2, WpT)


BN = 640


def _node1_body(h_ref, s_ref, deg_ref, WaT_ref, WbT_ref, bl_ref, o_ref):
    sacc = s_ref[...]
    deg = deg_ref[:, 0:1]
    hN = sacc / jnp.maximum(deg, 1.0)
    o = (jnp.dot(h_ref[...], WaT_ref[...], preferred_element_type=F32)
         + jnp.dot(hN, WbT_ref[...], preferred_element_type=F32)
         + bl_ref[...])
    o_ref[...] = jnp.maximum(o, 0.0)


def _node1_call(h, sacc, deg, WaT, WbT, bl):
    Dv = h.shape[1]
    Ho = WaT.shape[1]
    nb = lambda last: pl.BlockSpec((BN, last), lambda i: (i, 0))
    nb3 = lambda last: pl.BlockSpec((NC, BN, last), lambda i: (0, i, 0))
    return pl.pallas_call(
        _node1_body,
        grid=(NP // BN,),
        in_specs=[nb(128), nb(128), nb(128), _full(WaT.shape),
                  _full(WbT.shape), _full(bl.shape)],
        out_specs=nb(Ho),
        out_shape=jax.ShapeDtypeStruct((NP, Ho), F32),
    )(h, sacc, deg, WaT, WbT, bl)


def _node2_body(h_ref, p_ref, deg_ref, WaT_ref, bl_ref, o_ref):
    pacc = p_ref[:, 0:16]
    deg = deg_ref[:, 0:1]
    o = (jnp.dot(h_ref[...], WaT_ref[...], preferred_element_type=F32)
         + pacc / jnp.maximum(deg, 1.0)
         + bl_ref[...])
    o_ref[...] = o


def _node2_call(h, pacc, deg, WaT, bl):
    Dv = h.shape[1]
    Ho = WaT.shape[1]
    nb = lambda last: pl.BlockSpec((BN, last), lambda i: (i, 0))
    nb3 = lambda last: pl.BlockSpec((NC, BN, last), lambda i: (0, i, 0))
    return pl.pallas_call(
        _node2_body,
        grid=(NP // BN,),
        in_specs=[nb(Dv), nb(128), nb(128), _full(WaT.shape),
                  _full(bl.shape)],
        out_specs=nb(Ho),
        out_shape=jax.ShapeDtypeStruct((NP, Ho), F32),
    )(h, pacc, deg, WaT, bl)


_gather1 = _make_gather(128)
_gather2 = _make_gather(256)
_scatter1 = _make_scatter1()
_scatter_p2 = _make_scatter_p2()


def kernel(node_feat, edge_feat, edge_index, c1_We, c1_be, c1_Wr1, c1_br1,
           c1_Wr2, c1_br2, c1_Wl, c1_bl, c2_We, c2_be, c2_Wr1, c2_br1,
           c2_Wr2, c2_br2, c2_Wl, c2_bl):
    src = edge_index[0]
    dst = edge_index[1]
    D = node_feat.shape[1]
    H = c1_Wl.shape[0]

    hpad = jnp.pad(node_feat, ((0, NP - N), (0, 0)))

    hs1, hd1 = _gather1(hpad, src, dst)
    r1, w1 = _edge1_call(
        edge_feat, hs1, hd1,
        c1_We.T, c1_be.reshape(1, -1),
        c1_Wr1.T, c1_br1.reshape(1, -1),
        c1_Wr2.T, c1_br2.reshape(1, -1))
    s1, deg = _scatter1(r1, dst)
    h1 = _node1_call(hpad, s1, deg,
                     c1_Wl[:, :D].T, c1_Wl[:, D:].T,
                     c1_bl.reshape(1, -1))

    hs2, hd2 = _gather2(h1, src, dst)
    p2 = _edge2_call(
        w1, hs2, hd2,
        c2_We.T, c2_be.reshape(1, -1),
        c2_Wr1.T, c2_br1.reshape(1, -1),
        c2_Wr2.T, c2_br2.reshape(1, -1),
        jnp.pad(c2_Wl[:, H:].T, ((0, 0), (0, 112))))
    (sp2,) = _scatter_p2(p2, dst)
    out = _node2_call(h1, sp2, deg,
                      c2_Wl[:, :H].T, c2_bl.reshape(1, -1))
    return out[:N]


# speedup vs baseline: 2.1983x; 1.1807x over previous
import functools

import jax
import jax.numpy as jnp
from jax import lax
from jax.experimental import pallas as pl
from jax.experimental.pallas import tpu as pltpu
from jax.experimental.pallas import tpu_sc as plsc

N = 10000
NP = 10240
E = 320000
NC = 2
NS = 16
NW = NC * NS
GB = 80
RT = NP // NS
EW = E // NW
IT = EW // GB

F32 = jnp.float32


def _zero_fill(buf, width):
    def zrow(i, carry):
        def zcol(j, carry2):
            buf[i, pl.ds(j * 16, 16)] = jnp.zeros((16,), F32)
            return carry2
        return lax.fori_loop(0, width // 16, zcol, carry)
    lax.fori_loop(0, buf.shape[0], zrow, 0)


def _ones_fill(buf, width):
    def orow(i, carry):
        def ocol(j, carry2):
            buf[i, pl.ds(j * 16, 16)] = jnp.ones((16,), F32)
            return carry2
        return lax.fori_loop(0, width // 16, ocol, carry)
    lax.fori_loop(0, buf.shape[0], orow, 0)


def _make_gather(Dv):
    mesh = plsc.VectorSubcoreMesh(
        core_axis_name="c", subcore_axis_name="s",
        num_cores=NC, num_subcores=NS)
    GBG = 40
    ITG = EW // GBG
    HIT = ITG // 2

    @functools.partial(
        pl.kernel, mesh=mesh,
        out_type=[jax.ShapeDtypeStruct((E, Dv), F32),
                  jax.ShapeDtypeStruct((E, Dv), F32)],
        scratch_types=[pltpu.VMEM((2, GBG), jnp.int32),
                       pltpu.VMEM((2, GBG), jnp.int32),
                       pltpu.VMEM((GBG, Dv), F32),
                       pltpu.VMEM((GBG, Dv), F32),
                       pltpu.VMEM((GBG, Dv), F32),
                       pltpu.VMEM((GBG, Dv), F32),
                       pltpu.SemaphoreType.DMA,
                       pltpu.SemaphoreType.DMA,
                       pltpu.SemaphoreType.DMA,
                       pltpu.SemaphoreType.DMA,
                       pltpu.SemaphoreType.DMA,
                       pltpu.SemaphoreType.DMA,
                       pltpu.SemaphoreType.DMA,
                       pltpu.SemaphoreType.DMA],
    )
    def k(table, src_i, dst_i, hs_out, hd_out, SI, DI, rs0, rd0, rs1, rd1,
          gs0, gd0, gs1, gd1, ws0, wd0, ws1, wd1):
        wid = lax.axis_index("s") * NC + lax.axis_index("c")
        base = wid * EW

        def g(i, b):
            rs, rd = (rs0, rd0) if b == 0 else (rs1, rd1)
            gs, gd = (gs0, gd0) if b == 0 else (gs1, gd1)
            pltpu.async_copy(table.at[SI.at[b]], rs, gs)
            pltpu.async_copy(table.at[DI.at[b]], rd, gd)

        def wait_g(b):
            rs, rd = (rs0, rd0) if b == 0 else (rs1, rd1)
            gs, gd = (gs0, gd0) if b == 0 else (gs1, gd1)
            pltpu.make_async_copy(table.at[SI.at[b]], rs, gs).wait()
            pltpu.make_async_copy(table.at[DI.at[b]], rd, gd).wait()

        def w(e0, b):
            rs, rd = (rs0, rd0) if b == 0 else (rs1, rd1)
            ws, wd = (ws0, wd0) if b == 0 else (ws1, wd1)
            pltpu.async_copy(rs, hs_out.at[pl.ds(e0, GBG)], ws)
            pltpu.async_copy(rd, hd_out.at[pl.ds(e0, GBG)], wd)

        def wait_w(e0, b):
            rs, rd = (rs0, rd0) if b == 0 else (rs1, rd1)
            ws, wd = (ws0, wd0) if b == 0 else (ws1, wd1)
            pltpu.make_async_copy(rs, hs_out.at[pl.ds(e0, GBG)], ws).wait()
            pltpu.make_async_copy(rd, hd_out.at[pl.ds(e0, GBG)], wd).wait()

        def load_idx(i, b):
            e0 = base + i * GBG
            pltpu.sync_copy(src_i.at[pl.ds(e0, GBG)], SI.at[b])
            pltpu.sync_copy(dst_i.at[pl.ds(e0, GBG)], DI.at[b])

        load_idx(0, 0)
        g(0, 0)

        def body(kk, carry):
            i0 = 2 * kk
            e00 = base + i0 * GBG
            e01 = base + (i0 + 1) * GBG
            load_idx(i0 + 1, 1)

            @pl.when(kk > 0)
            def _():
                wait_w(base + (i0 - 1) * GBG, 1)
            wait_g(0)
            g(i0 + 1, 1)
            w(e00, 0)

            @pl.when(kk < HIT - 1)
            def _():
                load_idx(i0 + 2, 0)
            wait_w(e00, 0)
            wait_g(1)

            @pl.when(kk < HIT - 1)
            def _():
                g(i0 + 2, 0)
            w(e01, 1)
            return carry

        lax.fori_loop(0, HIT, body, 0)
        wait_w(base + (ITG - 1) * GBG, 1)

    return k


HALF = NP // 2
APAD = HALF + 128
ZR = APAD // NS
DR = HALF // NS


def _sweep_pipelined(dst, IDX, rb0, rb1, a0, a1, acc, lo, rng, data, base,
                     it_half, gb):
    def load(i, b):
        e0 = base + i * gb
        pltpu.sync_copy(dst.at[pl.ds(e0, gb)], IDX.at[b])
        if data is not None:
            rb = rb0 if b == 0 else rb1
            pltpu.sync_copy(data.at[pl.ds(e0, gb)], rb)
        for q in range(gb // 16):
            v = IDX[b, pl.ds(q * 16, 16)] - lo
            oob = (v < 0) | (v >= rng)
            IDX[b, pl.ds(q * 16, 16)] = jnp.where(oob, rng, v)

    def add(b):
        rb = rb0 if b == 0 else rb1
        sem = a0 if b == 0 else a1
        pltpu.async_copy(rb, acc.at[IDX.at[b]], sem, add=True)

    def wait_add(b):
        rb = rb0 if b == 0 else rb1
        sem = a0 if b == 0 else a1
        pltpu.make_async_copy(rb, acc.at[IDX.at[b]], sem).wait()

    load(0, 0)

    def body(kk, carry):
        i0 = 2 * kk
        add(0)

        @pl.when(kk > 0)
        def _():
            wait_add(1)
        load(i0 + 1, 1)
        add(1)

        @pl.when(kk < it_half - 1)
        def _():
            wait_add(0)
            load(i0 + 2, 0)
        return carry

    lax.fori_loop(0, it_half, body, 0)
    wait_add(0)
    wait_add(1)


def _make_scatter1():
    mesh = plsc.VectorSubcoreMesh(
        core_axis_name="c", subcore_axis_name="s",
        num_cores=NC, num_subcores=NS)
    EW1 = E // NS
    IT1 = EW1 // GB
    HIT1 = IT1 // 2

    @functools.partial(
        pl.kernel, mesh=mesh,
        out_type=[jax.ShapeDtypeStruct((NP, 128), F32),
                  jax.ShapeDtypeStruct((NP, 128), F32)],
        scratch_types=[pltpu.VMEM((2, GB), jnp.int32),
                       pltpu.VMEM((GB, 128), F32),
                       pltpu.VMEM((GB, 128), F32),
                       pltpu.VMEM((ZR, 128), F32),
                       pltpu.VMEM_SHARED((APAD, 128), F32),
                       pltpu.SemaphoreType.DMA,
                       pltpu.SemaphoreType.DMA],
    )
    def k(r, dst, s_out, deg_out, IDX, rb0, rb1, zbuf, acc, a0, a1):
        c = lax.axis_index("c")
        s = lax.axis_index("s")
        base = s * EW1
        lo = c * HALF

        _zero_fill(zbuf, 128)
        pltpu.sync_copy(zbuf, acc.at[pl.ds(s * ZR, ZR)])
        plsc.subcore_barrier()
        _sweep_pipelined(dst, IDX, rb0, rb1, a0, a1, acc, lo, HALF, r,
                         base, HIT1, GB)
        plsc.subcore_barrier()
        pltpu.sync_copy(acc.at[pl.ds(s * DR, DR)],
                        s_out.at[pl.ds(lo + s * DR, DR)])
        plsc.subcore_barrier()

        pltpu.sync_copy(zbuf, acc.at[pl.ds(s * ZR, ZR)])
        _ones_fill(rb0, 128)
        _ones_fill(rb1, 128)
        plsc.subcore_barrier()
        _sweep_pipelined(dst, IDX, rb0, rb1, a0, a1, acc, lo, HALF, None,
                         base, HIT1, GB)
        plsc.subcore_barrier()
        pltpu.sync_copy(acc.at[pl.ds(s * DR, DR)],
                        deg_out.at[pl.ds(lo + s * DR, DR)])

    return k


QTR = NP // 4
QPAD = QTR + 128
QZR = QPAD // NS
QDR = QTR // NS


def _make_scatter_p2():
    mesh = plsc.VectorSubcoreMesh(
        core_axis_name="c", subcore_axis_name="s",
        num_cores=NC, num_subcores=NS)
    EW1 = E // NS
    IT1 = EW1 // GB
    HIT1 = IT1 // 2

    @functools.partial(
        pl.kernel, mesh=mesh,
        out_type=[jax.ShapeDtypeStruct((NP, 128), F32)],
        scratch_types=[pltpu.VMEM((2, GB), jnp.int32),
                       pltpu.VMEM((GB, 128), F32),
                       pltpu.VMEM((GB, 128), F32),
                       pltpu.VMEM((QZR, 128), F32),
                       pltpu.VMEM_SHARED((QPAD, 128), F32),
                       pltpu.SemaphoreType.DMA,
                       pltpu.SemaphoreType.DMA],
    )
    def k(p, dst, s_out, IDX, rb0, rb1, zbuf, acc, a0, a1):
        c = lax.axis_index("c")
        s = lax.axis_index("s")
        base = s * EW1

        _zero_fill(zbuf, 128)
        for j in range(2):
            lo = (2 * c + j) * QTR
            pltpu.sync_copy(zbuf, acc.at[pl.ds(s * QZR, QZR)])
            plsc.subcore_barrier()
            _sweep_pipelined(dst, IDX, rb0, rb1, a0, a1, acc, lo, QTR, p,
                             base, HIT1, GB)
            plsc.subcore_barrier()
            pltpu.sync_copy(acc.at[pl.ds(s * QDR, QDR)],
                            s_out.at[pl.ds(lo + s * QDR, QDR)])
            plsc.subcore_barrier()

    return k


BE = 2000


def _cos_mlp(hs, hd, w, W1T, b1, W2T, b2):
    ns = jnp.maximum(jnp.sqrt(jnp.sum(hs * hs, axis=1, keepdims=True)), 1e-12)
    nd = jnp.maximum(jnp.sqrt(jnp.sum(hd * hd, axis=1, keepdims=True)), 1e-12)
    cos = jnp.sum(hs * hd, axis=1, keepdims=True) / (ns * nd)
    m = cos * w * hs
    rr = jnp.maximum(jnp.dot(m, W1T, preferred_element_type=F32) + b1, 0.0)
    rr = jnp.maximum(jnp.dot(rr, W2T, preferred_element_type=F32) + b2, 0.0)
    return rr


def _edge1_body(ef_ref, hs_ref, hd_ref, WeT_ref, be_ref, W1T_ref, b1_ref,
                W2T_ref, b2_ref, r_ref, w_ref):
    ef = ef_ref[...]
    WeT = WeT_ref[...]
    w = (ef[:, 0:1] * WeT[0:1, :] + ef[:, 1:2] * WeT[1:2, :]) + be_ref[...]
    r_ref[...] = _cos_mlp(hs_ref[...], hd_ref[...], w,
                          W1T_ref[...], b1_ref[...], W2T_ref[...], b2_ref[...])
    w_ref[...] = w


def _edge2_body(ef_ref, hs_ref, hd_ref, WeT_ref, be_ref, W1T_ref, b1_ref,
                W2T_ref, b2_ref, WpT_ref, p_ref):
    ef = jnp.maximum(ef_ref[...], 0.0)
    w = jnp.dot(ef, WeT_ref[...], preferred_element_type=F32) + be_ref[...]
    rr = _cos_mlp(hs_ref[...], hd_ref[...], w,
                  W1T_ref[...], b1_ref[...], W2T_ref[...], b2_ref[...])
    p_ref[...] = jnp.dot(rr, WpT_ref[...], preferred_element_type=F32)


def _full(shape):
    return pl.BlockSpec(shape, lambda i: (0,) * len(shape))


def _edge1_call(ef, hs, hd, WeT, be, W1T, b1, W2T, b2):
    Dv = hs.shape[1]
    eb = lambda last: pl.BlockSpec((BE, last), lambda i: (i, 0))
    return pl.pallas_call(
        _edge1_body,
        grid=(E // BE,),
        in_specs=[eb(2), eb(Dv), eb(Dv), _full(WeT.shape), _full(be.shape),
                  _full(W1T.shape), _full(b1.shape), _full(W2T.shape),
                  _full(b2.shape)],
        out_specs=[eb(Dv), eb(Dv)],
        out_shape=[jax.ShapeDtypeStruct((E, Dv), F32),
                   jax.ShapeDtypeStruct((E, Dv), F32)],
    )(ef, hs, hd, WeT, be, W1T, b1, W2T, b2)


def _edge2_call(ef, hs, hd, WeT, be, W1T, b1, W2T, b2, WpT):
    Dv = hs.shape[1]
    Din = ef.shape[1]
    Po = WpT.shape[1]
    eb = lambda last: pl.BlockSpec((BE, last), lambda i: (i, 0))
    return pl.pallas_call(
        _edge2_body,
        grid=(E // BE,),
        in_specs=[eb(Din), eb(Dv), eb(Dv), _full(WeT.shape), _full(be.shape),
                  _full(W1T.shape), _full(b1.shape), _full(W2T.shape),
                  _full(b2.shape), _full(WpT.shape)],
        out_specs=eb(Po),
        out_shape=jax.ShapeDtypeStruct((E, Po), F32),
    )(ef, hs, hd, WeT, be, W1T, b1, W2T, b2, WpT)


BN = 640


def _node1_body(h_ref, s_ref, deg_ref, WaT_ref, WbT_ref, bl_ref, o_ref):
    sacc = s_ref[...]
    deg = deg_ref[:, 0:1]
    hN = sacc / jnp.maximum(deg, 1.0)
    o = (jnp.dot(h_ref[...], WaT_ref[...], preferred_element_type=F32)
         + jnp.dot(hN, WbT_ref[...], preferred_element_type=F32)
         + bl_ref[...])
    o_ref[...] = jnp.maximum(o, 0.0)


def _node1_call(h, sacc, deg, WaT, WbT, bl):
    Dv = h.shape[1]
    Ho = WaT.shape[1]
    nb = lambda last: pl.BlockSpec((BN, last), lambda i: (i, 0))
    nb3 = lambda last: pl.BlockSpec((NC, BN, last), lambda i: (0, i, 0))
    return pl.pallas_call(
        _node1_body,
        grid=(NP // BN,),
        in_specs=[nb(128), nb(128), nb(128), _full(WaT.shape),
                  _full(WbT.shape), _full(bl.shape)],
        out_specs=nb(Ho),
        out_shape=jax.ShapeDtypeStruct((NP, Ho), F32),
    )(h, sacc, deg, WaT, WbT, bl)


def _node2_body(h_ref, p_ref, deg_ref, WaT_ref, bl_ref, o_ref):
    pacc = p_ref[:, 0:16]
    deg = deg_ref[:, 0:1]
    o = (jnp.dot(h_ref[...], WaT_ref[...], preferred_element_type=F32)
         + pacc / jnp.maximum(deg, 1.0)
         + bl_ref[...])
    o_ref[...] = o


def _node2_call(h, pacc, deg, WaT, bl):
    Dv = h.shape[1]
    Ho = WaT.shape[1]
    nb = lambda last: pl.BlockSpec((BN, last), lambda i: (i, 0))
    nb3 = lambda last: pl.BlockSpec((NC, BN, last), lambda i: (0, i, 0))
    return pl.pallas_call(
        _node2_body,
        grid=(NP // BN,),
        in_specs=[nb(Dv), nb(128), nb(128), _full(WaT.shape),
                  _full(bl.shape)],
        out_specs=nb(Ho),
        out_shape=jax.ShapeDtypeStruct((NP, Ho), F32),
    )(h, pacc, deg, WaT, bl)


_gather1 = _make_gather(128)
_gather2 = _make_gather(256)
_scatter1 = _make_scatter1()
_scatter_p2 = _make_scatter_p2()


def kernel(node_feat, edge_feat, edge_index, c1_We, c1_be, c1_Wr1, c1_br1,
           c1_Wr2, c1_br2, c1_Wl, c1_bl, c2_We, c2_be, c2_Wr1, c2_br1,
           c2_Wr2, c2_br2, c2_Wl, c2_bl):
    src = edge_index[0]
    dst = edge_index[1]
    D = node_feat.shape[1]
    H = c1_Wl.shape[0]

    hpad = jnp.pad(node_feat, ((0, NP - N), (0, 0)))

    hs1, hd1 = _gather1(hpad, src, dst)
    r1, w1 = _edge1_call(
        edge_feat, hs1, hd1,
        c1_We.T, c1_be.reshape(1, -1),
        c1_Wr1.T, c1_br1.reshape(1, -1),
        c1_Wr2.T, c1_br2.reshape(1, -1))
    s1, deg = _scatter1(r1, dst)
    h1 = _node1_call(hpad, s1, deg,
                     c1_Wl[:, :D].T, c1_Wl[:, D:].T,
                     c1_bl.reshape(1, -1))

    hs2, hd2 = _gather2(h1, src, dst)
    p2 = _edge2_call(
        w1, hs2, hd2,
        c2_We.T, c2_be.reshape(1, -1),
        c2_Wr1.T, c2_br1.reshape(1, -1),
        c2_Wr2.T, c2_br2.reshape(1, -1),
        jnp.pad(c2_Wl[:, H:].T, ((0, 0), (0, 112))))
    (sp2,) = _scatter_p2(p2, dst)
    out = _node2_call(h1, sp2, deg,
                      c2_Wl[:, :H].T, c2_bl.reshape(1, -1))
    return out[:N]
